# Initial kernel scaffold; baseline (speedup 1.0000x reference)
import jax
import jax.numpy as jnp
from jax.experimental import pallas as pl


def kernel(fc1_w, fc1_b, fc2_w, fc2_b, fc31_w, fc31_b, fc32_w, fc32_b, fc4_w, fc4_b, fc5_w, fc5_b, fc6_w, fc6_b, x, eps_key):
    raise NotImplementedError("write your pallas kernel here")


# speedup vs baseline: 1.2023x; 1.2023x over previous
import jax
import jax.numpy as jnp
from jax.experimental import pallas as pl
from jax.experimental.pallas import tpu as pltpu

_PIXELS = 4096
_HIDDEN_P = 1280
_LATENT_P = 128
_LATENT = 4


def _vae_kernel(x_ref, w1_ref, b1_ref, w2_ref, b2_ref, w31_ref, b31_ref,
                w32_ref, b32_ref, eps_ref, w4_ref, b4_ref, w5_ref, b5_ref,
                w6_ref, b6_ref, recon_ref, mu_ref, lv_ref, z_ref):
    f32 = jnp.float32
    xb = x_ref[...].astype(jnp.bfloat16)
    h = jnp.dot(xb, w1_ref[...], preferred_element_type=f32) + b1_ref[...]
    h = jnp.maximum(h, 0.0).astype(jnp.bfloat16)
    h = jnp.dot(h, w2_ref[...], preferred_element_type=f32) + b2_ref[...]
    h = jnp.maximum(h, 0.0).astype(jnp.bfloat16)
    mu = jnp.dot(h, w31_ref[...], preferred_element_type=f32) + b31_ref[...]
    lv = jnp.dot(h, w32_ref[...], preferred_element_type=f32) + b32_ref[...]
    z = mu + eps_ref[...] * jnp.exp(0.5 * lv)
    h4 = jnp.dot(z.astype(jnp.bfloat16), w4_ref[...],
                 preferred_element_type=f32) + b4_ref[...]
    h4 = jnp.maximum(h4, 0.0).astype(jnp.bfloat16)
    h5 = jnp.dot(h4, w5_ref[...], preferred_element_type=f32) + b5_ref[...]
    h5 = jnp.maximum(h5, 0.0).astype(jnp.bfloat16)
    recon = jnp.dot(h5, w6_ref[...], preferred_element_type=f32) + b6_ref[...]
    recon_ref[...] = jax.nn.sigmoid(recon)
    mu_ref[...] = mu
    lv_ref[...] = lv
    z_ref[...] = z


def kernel(fc1_w, fc1_b, fc2_w, fc2_b, fc31_w, fc31_b, fc32_w, fc32_b,
           fc4_w, fc4_b, fc5_w, fc5_b, fc6_w, fc6_b, x, eps_key):
    B = x.shape[0]
    x2 = x.reshape(B, _PIXELS)

    eps = jax.random.normal(eps_key, (B, _LATENT), dtype=jnp.float32)
    eps_p = jnp.pad(eps, ((0, 0), (0, _LATENT_P - _LATENT)))

    bm = 256
    while B % bm:
        bm //= 2
    grid = (B // bm,)

    row = lambda i: (i, 0)
    const = lambda i: (0, 0)

    recon, mu, lv, z = pl.pallas_call(
        _vae_kernel,
        out_shape=(
            jax.ShapeDtypeStruct((B, _PIXELS), jnp.float32),
            jax.ShapeDtypeStruct((B, _LATENT_P), jnp.float32),
            jax.ShapeDtypeStruct((B, _LATENT_P), jnp.float32),
            jax.ShapeDtypeStruct((B, _LATENT_P), jnp.float32),
        ),
        grid=grid,
        in_specs=[
            pl.BlockSpec((bm, _PIXELS), row),
            pl.BlockSpec((_PIXELS, _HIDDEN_P), const),
            pl.BlockSpec((1, _HIDDEN_P), const),
            pl.BlockSpec((_HIDDEN_P, _HIDDEN_P), const),
            pl.BlockSpec((1, _HIDDEN_P), const),
            pl.BlockSpec((_HIDDEN_P, _LATENT_P), const),
            pl.BlockSpec((1, _LATENT_P), const),
            pl.BlockSpec((_HIDDEN_P, _LATENT_P), const),
            pl.BlockSpec((1, _LATENT_P), const),
            pl.BlockSpec((bm, _LATENT_P), row),
            pl.BlockSpec((_LATENT_P, _HIDDEN_P), const),
            pl.BlockSpec((1, _HIDDEN_P), const),
            pl.BlockSpec((_HIDDEN_P, _HIDDEN_P), const),
            pl.BlockSpec((1, _HIDDEN_P), const),
            pl.BlockSpec((_HIDDEN_P, _PIXELS), const),
            pl.BlockSpec((1, _PIXELS), const),
        ],
        out_specs=(
            pl.BlockSpec((bm, _PIXELS), row),
            pl.BlockSpec((bm, _LATENT_P), row),
            pl.BlockSpec((bm, _LATENT_P), row),
            pl.BlockSpec((bm, _LATENT_P), row),
        ),
        compiler_params=pltpu.CompilerParams(
            dimension_semantics=("parallel",),
            vmem_limit_bytes=56 * 1024 * 1024,
        ),
    )(x2, fc1_w, fc1_b, fc2_w, fc2_b, fc31_w, fc31_b, fc32_w, fc32_b,
      eps_p, fc4_w, fc4_b, fc5_w, fc5_b, fc6_w, fc6_b)

    return recon, mu[:, :_LATENT], lv[:, :_LATENT], z[:, :_LATENT]
